# Initial kernel scaffold; baseline (speedup 1.0000x reference)
import jax
import jax.numpy as jnp
from jax.experimental import pallas as pl


def kernel(x, edge_index, edge_attr, ne_W1, ne_b1, ne_W2, ne_b2, ee_W1, ee_b1, ee_W2, ee_b2, conv_W1, conv_b1, conv_W2, conv_b2, bn_g, bn_b, ffn_W1, ffn_b1, ffn_W2, ffn_b2, ln_g, ln_b, sg_W, sg_b, dg_W, dg_b, eg_W, eg_b, h_W1, h_b1, h_W2, h_b2, h_W3, h_b3):
    raise NotImplementedError("write your pallas kernel here")


# speedup vs baseline: 2.3730x; 2.3730x over previous
import functools

import jax
import jax.numpy as jnp
from jax import lax
from jax.experimental import pallas as pl
from jax.experimental.pallas import tpu as pltpu
from jax.experimental.pallas import tpu_sc as plsc

N = 10000
E = 320000
D = 128
DE = 16
H = 64
HP = 128
L = 3

NC = 2
NS = 16
NW = NC * NS

CHUNK = 80
CHUNKS_TOTAL = E // CHUNK
CPW = CHUNKS_TOTAL // NW
NPAD = 10240
ROWS_PER_S = NPAD // NS
ZROWS = 128

GB = 400
GPW = E // NW // GB

BE = 2560
NEB = E // BE


def _msg_body(h_hbm, e_hbm, src_hbm, dst_hbm, out_hbm,
              src_v, dst_v, rows_v, e_v, zbuf_v, sem, acc):
    c = lax.axis_index("c")
    s = lax.axis_index("s")
    wid = s * NC + c

    def _zrow(i, carry):
        for k in range(HP // 16):
            zbuf_v[i, pl.ds(k * 16, 16)] = jnp.zeros((16,), jnp.float32)
        return carry
    lax.fori_loop(0, ZROWS, _zrow, 0)
    for k in range(ROWS_PER_S // ZROWS):
        pltpu.sync_copy(zbuf_v, acc.at[pl.ds(s * ROWS_PER_S + k * ZROWS, ZROWS)])
    plsc.subcore_barrier()

    def _chunk(j, carry):
        eoff = wid * (CPW * CHUNK) + j * CHUNK
        pltpu.sync_copy(src_hbm.at[pl.ds(eoff, CHUNK)], src_v)
        gather = pltpu.async_copy(h_hbm.at[src_v], rows_v, sem)
        pltpu.sync_copy(dst_hbm.at[pl.ds(eoff, CHUNK)], dst_v)
        pltpu.sync_copy(
            e_hbm.at[pl.ds(wid * (CPW * CHUNK // 2) + j * (CHUNK // 2),
                           CHUNK // 2)], e_v)
        gather.wait()

        def _row(i2, carry2):
            for p in range(2):
                for k in range(H // 16):
                    sl = pl.ds(k * 16, 16)
                    esl = pl.ds(p * H + k * 16, 16)
                    rows_v[2 * i2 + p, sl] = jnp.maximum(
                        rows_v[2 * i2 + p, sl] + e_v[i2, esl], 0.0)
            return carry2
        lax.fori_loop(0, CHUNK // 2, _row, 0)
        pltpu.sync_copy(rows_v, acc.at[dst_v], add=True)
        return carry
    lax.fori_loop(0, CPW, _chunk, 0)

    plsc.subcore_barrier()
    for k in range(ROWS_PER_S // ZROWS):
        r0 = s * ROWS_PER_S + k * ZROWS
        pltpu.sync_copy(acc.at[pl.ds(r0, ZROWS)], zbuf_v)
        pltpu.sync_copy(zbuf_v, out_hbm.at[c, pl.ds(r0, ZROWS)])


@functools.cache
def _msg_call():
    return pl.kernel(
        _msg_body,
        out_type=jax.ShapeDtypeStruct((NC, NPAD, HP), jnp.float32),
        mesh=plsc.VectorSubcoreMesh(core_axis_name="c", subcore_axis_name="s",
                                    num_cores=NC, num_subcores=NS),
        scratch_types=[
            pltpu.VMEM((CHUNK,), jnp.int32),
            pltpu.VMEM((CHUNK,), jnp.int32),
            pltpu.VMEM((CHUNK, HP), jnp.float32),
            pltpu.VMEM((CHUNK // 2, HP), jnp.float32),
            pltpu.VMEM((ZROWS, HP), jnp.float32),
            pltpu.SemaphoreType.DMA,
            pltpu.VMEM_SHARED((NPAD, HP), jnp.float32),
        ],
    )


def _gather_body(h_hbm, src_hbm, dst_hbm, out_hbm, idx_v, rows_s, rows_d, sem):
    c = lax.axis_index("c")
    s = lax.axis_index("s")
    wid = s * NC + c
    base = wid * (E // NW)

    def _chunk(j, carry):
        off = base + j * GB
        pltpu.sync_copy(src_hbm.at[pl.ds(off, GB)], idx_v)
        pltpu.async_copy(h_hbm.at[idx_v], rows_s, sem).wait()
        pltpu.sync_copy(dst_hbm.at[pl.ds(off, GB)], idx_v)
        pltpu.async_copy(h_hbm.at[idx_v], rows_d, sem).wait()

        def _row(i, carry2):
            for k in range(H // 16):
                rows_s[i, pl.ds(H + k * 16, 16)] = rows_d[i, pl.ds(k * 16, 16)]
            return carry2
        lax.fori_loop(0, GB, _row, 0)
        pltpu.sync_copy(rows_s, out_hbm.at[pl.ds(off, GB)])
        return carry
    lax.fori_loop(0, GPW, _chunk, 0)


@functools.cache
def _gather_call():
    return pl.kernel(
        _gather_body,
        out_type=jax.ShapeDtypeStruct((E, HP), jnp.float32),
        mesh=plsc.VectorSubcoreMesh(core_axis_name="c", subcore_axis_name="s",
                                    num_cores=NC, num_subcores=NS),
        scratch_types=[
            pltpu.VMEM((GB,), jnp.int32),
            pltpu.VMEM((GB, HP), jnp.float32),
            pltpu.VMEM((GB, HP), jnp.float32),
            pltpu.SemaphoreType.DMA,
        ],
    )


def _mlp2_body(x_ref, W1_ref, b1_ref, W2_ref, b2_ref, out_ref):
    z = jnp.maximum(jnp.dot(x_ref[...], W1_ref[...],
                            preferred_element_type=jnp.float32) + b1_ref[...], 0.0)
    out_ref[...] = jnp.dot(z, W2_ref[...],
                           preferred_element_type=jnp.float32) + b2_ref[...]


def _mlp2_pad_body(x_ref, W1_ref, b1_ref, W2_ref, b2_ref, out_ref):
    z = jnp.maximum(jnp.dot(x_ref[...], W1_ref[...],
                            preferred_element_type=jnp.float32) + b1_ref[...], 0.0)
    r = jnp.dot(z, W2_ref[...], preferred_element_type=jnp.float32) + b2_ref[...]
    out_ref[:, :H] = r
    out_ref[:, H:] = jnp.zeros_like(r)


def _edge_enc_body(ea_ref, W1_ref, b1_ref, W2_ref, b2_ref, out_ref):
    ea = ea_ref[...]
    prior = (1.5 * ea[:, 0] + 0.7 * ea[:, 1] + 0.6 * ea[:, 2] + 0.5 * ea[:, 3]
             - 0.9 * ea[:, 4] - 0.7 * ea[:, 5] - 0.45 * ea[:, 6]
             + 0.15 * ea[:, 7])[:, None]
    ea_aug = jnp.concatenate([ea, prior], axis=-1)
    z = jnp.maximum(jnp.dot(ea_aug, W1_ref[...],
                            preferred_element_type=jnp.float32) + b1_ref[...], 0.0)
    out_ref[...] = jnp.dot(z, W2_ref[...],
                           preferred_element_type=jnp.float32) + b2_ref[...]


def _edge_encode(ea, W1, b1, W2, b2):
    return pl.pallas_call(
        _edge_enc_body,
        grid=(NEB,),
        in_specs=[
            pl.BlockSpec((BE, DE), lambda i: (i, 0)),
            pl.BlockSpec((DE + 1, H), lambda i: (0, 0)),
            pl.BlockSpec((1, H), lambda i: (0, 0)),
            pl.BlockSpec((H, H), lambda i: (0, 0)),
            pl.BlockSpec((1, H), lambda i: (0, 0)),
        ],
        out_specs=pl.BlockSpec((BE, H), lambda i: (i, 0)),
        out_shape=jax.ShapeDtypeStruct((E, H), jnp.float32),
    )(ea, W1, b1, W2, b2)


def _node_encode(x, W1, b1, W2, b2):
    return pl.pallas_call(
        _mlp2_pad_body,
        out_shape=jax.ShapeDtypeStruct((N, HP), jnp.float32),
    )(x, W1, b1, W2, b2)


def _layer_pre_body(h_ref, agg_ref, cW1, cb1, cW2, cb2, out_ref):
    h = h_ref[:, :H]
    z = h + agg_ref[0][:N, :H] + agg_ref[1][:N, :H]
    z = jnp.maximum(jnp.dot(z, cW1[...], preferred_element_type=jnp.float32)
                    + cb1[...], 0.0)
    out_ref[...] = jnp.dot(z, cW2[...], preferred_element_type=jnp.float32) + cb2[...]


def _layer_post_body(h_ref, z_ref, mu_ref, var_ref, bng, bnb,
                     fW1, fb1, fW2, fb2, lng, lnb, out_ref):
    h = h_ref[:, :H]
    z = ((z_ref[...] - mu_ref[...]) / jnp.sqrt(var_ref[...] + 1e-5)
         * bng[...] + bnb[...])
    h = h + jnp.maximum(z, 0.0)
    y = jnp.maximum(jnp.dot(h, fW1[...], preferred_element_type=jnp.float32)
                    + fb1[...], 0.0)
    y = jnp.dot(y, fW2[...], preferred_element_type=jnp.float32) + fb2[...]
    t = h + y
    mu = jnp.mean(t, axis=-1, keepdims=True)
    var = jnp.mean((t - mu) * (t - mu), axis=-1, keepdims=True)
    out_ref[:, :H] = (t - mu) / jnp.sqrt(var + 1e-5) * lng[...] + lnb[...]
    out_ref[:, H:] = jnp.zeros_like(t)


def _layer_update(h, agg, cW1, cb1, cW2, cb2, bng, bnb,
                  fW1, fb1, fW2, fb2, lng, lnb):
    z = pl.pallas_call(
        _layer_pre_body,
        out_shape=jax.ShapeDtypeStruct((N, H), jnp.float32),
    )(h, agg, cW1, cb1, cW2, cb2)
    mu = z.mean(axis=0, keepdims=True)
    var = z.var(axis=0, keepdims=True)
    return pl.pallas_call(
        _layer_post_body,
        out_shape=jax.ShapeDtypeStruct((N, HP), jnp.float32),
    )(h, z, mu, var, bng, bnb, fW1, fb1, fW2, fb2, lng, lnb)


def _head_body(xsd_ref, e_ref, sgW, dgW, egW, gb,
               W1a, W1b, W1c, W1d, hb1, hW2, hb2, hW3, hb3, out_ref):
    xs = xsd_ref[:, :H]
    xd = xsd_ref[:, H:]
    e = e_ref[...]
    g = jax.nn.sigmoid(
        jnp.dot(xs, sgW[...], preferred_element_type=jnp.float32)
        + jnp.dot(xd, dgW[...], preferred_element_type=jnp.float32)
        + jnp.dot(e, egW[...], preferred_element_type=jnp.float32) + gb[...])
    t = g * (xs * xd)
    hd = jnp.maximum(
        jnp.dot(xs, W1a[...], preferred_element_type=jnp.float32)
        + jnp.dot(xd, W1b[...], preferred_element_type=jnp.float32)
        + jnp.dot(e, W1c[...], preferred_element_type=jnp.float32)
        + jnp.dot(t, W1d[...], preferred_element_type=jnp.float32)
        + hb1[...], 0.0)
    hd = jnp.maximum(jnp.dot(hd, hW2[...], preferred_element_type=jnp.float32)
                     + hb2[...], 0.0)
    out_ref[...] = (jnp.dot(hd, hW3[...], preferred_element_type=jnp.float32)
                    + hb3[...])


def _head(xsd, e, sgW, dgW, egW, gb, W1a, W1b, W1c, W1d,
          hb1, hW2, hb2, hW3, hb3):
    wspec = lambda shape: pl.BlockSpec(shape, lambda i: (0,) * len(shape))
    return pl.pallas_call(
        _head_body,
        grid=(NEB,),
        in_specs=[
            pl.BlockSpec((BE, HP), lambda i: (i, 0)),
            pl.BlockSpec((BE, H), lambda i: (i, 0)),
            wspec((H, H)), wspec((H, H)), wspec((H, H)), wspec((1, H)),
            wspec((H, H)), wspec((H, H)), wspec((H, H)), wspec((H, H)),
            wspec((1, H)), wspec((H, H // 2)), wspec((1, H // 2)),
            wspec((H // 2, 1)), wspec((1, 1)),
        ],
        out_specs=pl.BlockSpec((BE, 1), lambda i: (i, 0)),
        out_shape=jax.ShapeDtypeStruct((E, 1), jnp.float32),
    )(xsd, e, sgW, dgW, egW, gb, W1a, W1b, W1c, W1d,
      hb1, hW2, hb2, hW3, hb3)


def kernel(x, edge_index, edge_attr,
           ne_W1, ne_b1, ne_W2, ne_b2,
           ee_W1, ee_b1, ee_W2, ee_b2,
           conv_W1, conv_b1, conv_W2, conv_b2,
           bn_g, bn_b,
           ffn_W1, ffn_b1, ffn_W2, ffn_b2,
           ln_g, ln_b,
           sg_W, sg_b, dg_W, dg_b, eg_W, eg_b,
           h_W1, h_b1, h_W2, h_b2, h_W3, h_b3):
    src = edge_index[0]
    dst = edge_index[1]

    r1 = lambda b: b.reshape(1, -1)

    e = _edge_encode(edge_attr, ee_W1, r1(ee_b1), ee_W2, r1(ee_b2))
    h = _node_encode(x, ne_W1, r1(ne_b1), ne_W2, r1(ne_b2))

    e2 = e.reshape(E // 2, HP)
    for l in range(L):
        agg = _msg_call()(h, e2, src, dst)
        h = _layer_update(h, agg,
                          conv_W1[l], r1(conv_b1[l]), conv_W2[l], r1(conv_b2[l]),
                          r1(bn_g[l]), r1(bn_b[l]),
                          ffn_W1[l], r1(ffn_b1[l]), ffn_W2[l], r1(ffn_b2[l]),
                          r1(ln_g[l]), r1(ln_b[l]))

    xsd = _gather_call()(h, src, dst)

    gb = r1(sg_b + dg_b + eg_b)
    logits = _head(xsd, e, sg_W, dg_W, eg_W, gb,
                   h_W1[0 * H:1 * H], h_W1[1 * H:2 * H],
                   h_W1[2 * H:3 * H], h_W1[3 * H:4 * H],
                   r1(h_b1), h_W2, r1(h_b2), h_W3, r1(h_b3))
    return logits[:, 0]
